# baseline (speedup 1.0000x reference)
import functools

import jax
import jax.numpy as jnp
from jax import lax
from jax.experimental import pallas as pl
from jax.experimental.pallas import tpu as pltpu
from jax.experimental.pallas import tpu_sc as plsc

_B, _T, _D, _NNEG = 2, 2048, 768, 10
_BT = _B * _T
_NR = _NNEG * _B * _T
_NC, _NS = 2, 16
_NW = _NC * _NS
_C = 40
_NBUF = 4
_NPW = _NR // _NW
_NCH = _NPW // _C


@functools.partial(
    pl.kernel,
    out_type=jax.ShapeDtypeStruct((_NR, _D), jnp.float32),
    mesh=plsc.VectorSubcoreMesh(core_axis_name="c", subcore_axis_name="s"),
    scratch_types=(
        pltpu.VMEM((_NPW,), jnp.int32),
        pltpu.VMEM((_C, _D), jnp.float32),
        pltpu.VMEM((_C, _D), jnp.float32),
        pltpu.VMEM((_C, _D), jnp.float32),
        pltpu.VMEM((_C, _D), jnp.float32),
        pltpu.SemaphoreType.DMA,
        pltpu.SemaphoreType.DMA,
        pltpu.SemaphoreType.DMA,
        pltpu.SemaphoreType.DMA,
    ),
)
def _sc_gather(x_hbm, idxn_hbm, neg_hbm, idxn_v,
               buf0, buf1, buf2, buf3, sem0, sem1, sem2, sem3):
    wid = lax.axis_index("s") * _NC + lax.axis_index("c")
    nbase = wid * _NPW
    bufs = (buf0, buf1, buf2, buf3)
    sems = (sem0, sem1, sem2, sem3)

    pltpu.sync_copy(idxn_hbm.at[pl.ds(nbase, _NPW)], idxn_v)

    def ngather(c, b):
        pltpu.async_copy(x_hbm.at[idxn_v.at[pl.ds(c * _C, _C)]], bufs[b], sems[b])

    def nwait(c, b):
        pltpu.make_async_copy(x_hbm.at[idxn_v.at[pl.ds(c * _C, _C)]],
                              bufs[b], sems[b]).wait()

    for b in range(_NBUF - 1):
        ngather(b, b)

    def nbody(k, carry):
        c = _NBUF * k
        for j in range(_NBUF):
            nxt = c + j + _NBUF - 1
            if j == 0:
                ngather(nxt, _NBUF - 1)
            else:
                @pl.when(nxt < _NCH)
                def _(nxt=nxt, b=j - 1):
                    ngather(nxt, b)
            nwait(c + j, j)
            pltpu.sync_copy(bufs[j], neg_hbm.at[pl.ds(nbase + (c + j) * _C, _C)])
        return carry

    lax.fori_loop(0, _NCH // _NBUF, nbody, 0)


def _tc_roll_body(x_ref, tgt_ref, xcopy_ref):
    tgt_ref[pl.ds(0, _BT - 1), :] = x_ref[pl.ds(1, _BT - 1), :]
    tgt_ref[pl.ds(_T - 1, 1), :] = x_ref[pl.ds(0, 1), :]
    tgt_ref[pl.ds(_BT - 1, 1), :] = x_ref[pl.ds(_T, 1), :]
    xcopy_ref[...] = x_ref[...]


_tc_roll = pl.pallas_call(
    _tc_roll_body,
    out_shape=(
        jax.ShapeDtypeStruct((_BT, _D), jnp.float32),
        jax.ShapeDtypeStruct((_BT, _D), jnp.float32),
    ),
)


def kernel(x):
    B, T, D = x.shape
    tszs = jnp.repeat(jnp.arange(T), _NNEG)
    neg = jax.random.randint(jax.random.key(42), (B, _NNEG * T), 0, T - 1)
    neg = jnp.where(neg >= tszs[None, :], neg + 1, neg)
    src_t = jnp.where(neg == T - 1, 0, neg + 1)
    src = src_t + jnp.arange(B)[:, None] * T
    idxn = src.reshape(B, T, _NNEG).transpose(2, 0, 1).reshape(-1)
    idxn = idxn.astype(jnp.int32)

    x_flat = x.reshape(_BT, D)
    negs = _sc_gather(x_flat, idxn)
    tgt, xc = _tc_roll(x_flat)
    return (xc.reshape(B, T, D), tgt.reshape(B, T, D),
            negs.reshape(_NNEG, B, T, D))
